# Initial kernel scaffold; baseline (speedup 1.0000x reference)
import jax
import jax.numpy as jnp
from jax.experimental import pallas as pl


def kernel(query, key, value, in_proj_weight, in_proj_bias, out_proj_w, out_proj_b, dgl_ln_g, dgl_ln_b, dgl_w1, dgl_b1, dgl_w2, dgl_b2):
    raise NotImplementedError("write your pallas kernel here")


# speedup vs baseline: 1.6458x; 1.6458x over previous
import functools
import math

import jax
import jax.numpy as jnp
from jax.experimental import pallas as pl
from jax.experimental.pallas import tpu as pltpu

H = 12

_DN = (((1,), (1,)), ((), ()))


def _qkv_proj_kernel(xq_ref, xk_ref, xv_ref, w_ref, b_ref, qo_ref, ko_ref, vo_ref):
    e = w_ref.shape[1]
    xq = xq_ref[...]
    xk = xk_ref[...]
    xv = xv_ref[...]
    qo_ref[...] = jax.lax.dot_general(xq, w_ref[:e, :], _DN, preferred_element_type=jnp.float32) + b_ref[0]
    ko_ref[...] = jax.lax.dot_general(xk, w_ref[e:2 * e, :], _DN, preferred_element_type=jnp.float32) + b_ref[1]
    vo_ref[...] = jax.lax.dot_general(xv, w_ref[2 * e:, :], _DN, preferred_element_type=jnp.float32) + b_ref[2]


def _attn_kernel(q_ref, k_ref, v_ref, ao_ref, aw_ref, *, scale, tq):
    h = pl.program_id(1)
    i = pl.program_id(2)
    q = q_ref[0] * scale
    s = jax.lax.dot_general(q, k_ref[0], _DN, preferred_element_type=jnp.float32)
    m = jnp.max(s, axis=-1, keepdims=True)
    e = jnp.exp(s - m)
    p = e / jnp.sum(e, axis=-1, keepdims=True)
    ao_ref[0] = jax.lax.dot_general(
        p, v_ref[0], (((1,), (0,)), ((), ())), preferred_element_type=jnp.float32)
    sl = pl.ds(i * tq, tq)

    @pl.when(h == 0)
    def _():
        aw_ref[0, sl, :] = p

    @pl.when(jnp.logical_and(h > 0, h < H - 1))
    def _():
        aw_ref[0, sl, :] = aw_ref[0, sl, :] + p

    @pl.when(h == H - 1)
    def _():
        aw_ref[0, sl, :] = (aw_ref[0, sl, :] + p) * (1.0 / H)


def _linear_kernel(x_ref, w_ref, b_ref, o_ref):
    o_ref[...] = jax.lax.dot_general(
        x_ref[...], w_ref[...], _DN, preferred_element_type=jnp.float32) + b_ref[0]


def kernel(query, key, value, in_proj_weight, in_proj_bias, out_proj_w, out_proj_b,
           dgl_ln_g, dgl_ln_b, dgl_w1, dgl_b1, dgl_w2, dgl_b2):
    T, B, E = query.shape
    S = key.shape[0]
    hd = E // H
    M = T * B

    mt = 512
    q, k, v = pl.pallas_call(
        _qkv_proj_kernel,
        grid=(M // mt,),
        in_specs=[
            pl.BlockSpec((mt, E), lambda m: (m, 0)),
            pl.BlockSpec((mt, E), lambda m: (m, 0)),
            pl.BlockSpec((mt, E), lambda m: (m, 0)),
            pl.BlockSpec((3 * E, E), lambda m: (0, 0)),
            pl.BlockSpec((3, E), lambda m: (0, 0)),
        ],
        out_specs=[pl.BlockSpec((mt, E), lambda m: (m, 0))] * 3,
        out_shape=[jax.ShapeDtypeStruct((M, E), jnp.float32)] * 3,
    )(query.reshape(M, E), key.reshape(M, E), value.reshape(M, E),
      in_proj_weight, in_proj_bias.reshape(3, E))

    def to_heads(x, L):
        return x.reshape(L, B, H, hd).transpose(1, 2, 0, 3).reshape(B * H, L, hd)

    qh = to_heads(q, T)
    kh = to_heads(k, S)
    vh = to_heads(v, S)

    tq = 256
    scale = 1.0 / math.sqrt(hd)
    ao, aw = pl.pallas_call(
        functools.partial(_attn_kernel, scale=scale, tq=tq),
        grid=(B, H, T // tq),
        in_specs=[
            pl.BlockSpec((1, tq, hd), lambda b, h, i: (b * H + h, i, 0)),
            pl.BlockSpec((1, S, hd), lambda b, h, i: (b * H + h, 0, 0)),
            pl.BlockSpec((1, S, hd), lambda b, h, i: (b * H + h, 0, 0)),
        ],
        out_specs=[
            pl.BlockSpec((1, tq, hd), lambda b, h, i: (b * H + h, i, 0)),
            pl.BlockSpec((1, T, S), lambda b, h, i: (b, 0, 0)),
        ],
        out_shape=[
            jax.ShapeDtypeStruct((B * H, T, hd), jnp.float32),
            jax.ShapeDtypeStruct((B, T, S), jnp.float32),
        ],
        compiler_params=pltpu.CompilerParams(
            dimension_semantics=("arbitrary", "arbitrary", "arbitrary")),
    )(qh, kh, vh)

    aoc = ao.transpose(1, 0, 2).reshape(M, E)
    out = pl.pallas_call(
        _linear_kernel,
        grid=(M // mt,),
        in_specs=[
            pl.BlockSpec((mt, E), lambda m: (m, 0)),
            pl.BlockSpec((E, E), lambda m: (0, 0)),
            pl.BlockSpec((1, E), lambda m: (0, 0)),
        ],
        out_specs=pl.BlockSpec((mt, E), lambda m: (m, 0)),
        out_shape=jax.ShapeDtypeStruct((M, E), jnp.float32),
    )(aoc, out_proj_w, out_proj_b.reshape(1, E))

    return out.reshape(T, B, E), aw


# speedup vs baseline: 2.0520x; 1.2469x over previous
import functools
import math

import jax
import jax.numpy as jnp
from jax.experimental import pallas as pl
from jax.experimental.pallas import tpu as pltpu

H = 12

_DNT = (((1,), (1,)), ((), ()))
_DN = (((1,), (0,)), ((), ()))


def _qkv_proj_kernel(xq_ref, xk_ref, xv_ref, w_ref, b_ref, qo_ref, ko_ref, vo_ref,
                     *, n_b, e, scale):
    hd = e // H
    for b in range(n_b):
        cols = slice(b * e, (b + 1) * e)
        xq = xq_ref[:, cols]
        xk = xk_ref[:, cols]
        xv = xv_ref[:, cols]
        yq = jax.lax.dot_general(xq, w_ref[:e, :], _DNT,
                                 preferred_element_type=jnp.float32) + b_ref[0]
        yk = jax.lax.dot_general(xk, w_ref[e:2 * e, :], _DNT,
                                 preferred_element_type=jnp.float32) + b_ref[1]
        yv = jax.lax.dot_general(xv, w_ref[2 * e:, :], _DNT,
                                 preferred_element_type=jnp.float32) + b_ref[2]
        yq = yq * scale
        for h in range(H):
            hs = slice(h * hd, (h + 1) * hd)
            qo_ref[b * H + h] = yq[:, hs]
            ko_ref[b * H + h] = yk[:, hs]
            vo_ref[b * H + h] = yv[:, hs]


def _attn_kernel(q_ref, k_ref, v_ref, wot_ref, ob_ref, out_ref, aw_ref, *, tq, e):
    h = pl.program_id(1)
    i = pl.program_id(2)
    b = pl.program_id(0)
    s = jax.lax.dot_general(q_ref[0], k_ref[0], _DNT,
                            preferred_element_type=jnp.float32)
    m = jnp.max(s, axis=-1, keepdims=True)
    ex = jnp.exp(s - m)
    p = ex / jnp.sum(ex, axis=-1, keepdims=True)
    pv = jax.lax.dot_general(p, v_ref[0], _DN,
                             preferred_element_type=jnp.float32)
    o = jax.lax.dot_general(pv, wot_ref[0], _DN,
                            preferred_element_type=jnp.float32)
    rows = pl.ds(i * tq, tq)
    cols = pl.ds(b * e, e)

    @pl.when(h == 0)
    def _():
        out_ref[rows, cols] = o
        aw_ref[0, rows, :] = p

    @pl.when(jnp.logical_and(h > 0, h < H - 1))
    def _():
        out_ref[rows, cols] = out_ref[rows, cols] + o
        aw_ref[0, rows, :] = aw_ref[0, rows, :] + p

    @pl.when(h == H - 1)
    def _():
        out_ref[rows, cols] = out_ref[rows, cols] + o + ob_ref[0]
        aw_ref[0, rows, :] = (aw_ref[0, rows, :] + p) * (1.0 / H)


def kernel(query, key, value, in_proj_weight, in_proj_bias, out_proj_w, out_proj_b,
           dgl_ln_g, dgl_ln_b, dgl_w1, dgl_b1, dgl_w2, dgl_b2):
    T, B, E = query.shape
    S = key.shape[0]
    hd = E // H
    scale = 1.0 / math.sqrt(hd)

    tt = 256
    qh, kh, vh = pl.pallas_call(
        functools.partial(_qkv_proj_kernel, n_b=B, e=E, scale=scale),
        grid=(T // tt,),
        in_specs=[
            pl.BlockSpec((tt, B * E), lambda i: (i, 0)),
            pl.BlockSpec((tt, B * E), lambda i: (i, 0)),
            pl.BlockSpec((tt, B * E), lambda i: (i, 0)),
            pl.BlockSpec((3 * E, E), lambda i: (0, 0)),
            pl.BlockSpec((3, E), lambda i: (0, 0)),
        ],
        out_specs=[pl.BlockSpec((B * H, tt, hd), lambda i: (0, i, 0))] * 3,
        out_shape=[jax.ShapeDtypeStruct((B * H, T, hd), jnp.float32)] * 3,
    )(query.reshape(T, B * E), key.reshape(S, B * E), value.reshape(S, B * E),
      in_proj_weight, in_proj_bias.reshape(3, E))

    wot = out_proj_w.T.reshape(H, hd, E)

    tq = 256
    out, aw = pl.pallas_call(
        functools.partial(_attn_kernel, tq=tq, e=E),
        grid=(B, H, T // tq),
        in_specs=[
            pl.BlockSpec((1, tq, hd), lambda b, h, i: (b * H + h, i, 0)),
            pl.BlockSpec((1, S, hd), lambda b, h, i: (b * H + h, 0, 0)),
            pl.BlockSpec((1, S, hd), lambda b, h, i: (b * H + h, 0, 0)),
            pl.BlockSpec((1, hd, E), lambda b, h, i: (h, 0, 0)),
            pl.BlockSpec((1, E), lambda b, h, i: (0, 0)),
        ],
        out_specs=[
            pl.BlockSpec((T, B * E), lambda b, h, i: (0, 0)),
            pl.BlockSpec((1, T, S), lambda b, h, i: (b, 0, 0)),
        ],
        out_shape=[
            jax.ShapeDtypeStruct((T, B * E), jnp.float32),
            jax.ShapeDtypeStruct((B, T, S), jnp.float32),
        ],
        compiler_params=pltpu.CompilerParams(
            dimension_semantics=("arbitrary", "arbitrary", "arbitrary")),
    )(qh, kh, vh, wot, out_proj_b.reshape(1, E))

    return out.reshape(T, B, E), aw


# speedup vs baseline: 2.2784x; 1.1103x over previous
import functools
import math

import jax
import jax.numpy as jnp
from jax.experimental import pallas as pl
from jax.experimental.pallas import tpu as pltpu

H = 12

_C00 = (((0,), (0,)), ((), ()))
_C11 = (((1,), (1,)), ((), ()))
_C10 = (((1,), (0,)), ((), ()))


def _qkv_proj_kernel(xq_ref, xk_ref, xv_ref, w_ref, b_ref, qo_ref, ko_ref, vo_ref,
                     *, n_b, e, scale):
    for b in range(n_b):
        xq = xq_ref[:, b, :]
        xk = xk_ref[:, b, :]
        xv = xv_ref[:, b, :]
        rows = slice(b * e, (b + 1) * e)
        qo_ref[rows, :] = (jax.lax.dot_general(
            w_ref[:e, :], xq, _C11, preferred_element_type=jnp.float32)
            + b_ref[:e, :]) * scale
        ko_ref[rows, :] = jax.lax.dot_general(
            w_ref[e:2 * e, :], xk, _C11, preferred_element_type=jnp.float32
        ) + b_ref[e:2 * e, :]
        vo_ref[rows, :] = jax.lax.dot_general(
            w_ref[2 * e:, :], xv, _C11, preferred_element_type=jnp.float32
        ) + b_ref[2 * e:, :]


def _attn_kernel(q_ref, k_ref, v_ref, wot_ref, ob_ref, out_ref, aw_ref, *, tq):
    h = pl.program_id(1)
    i = pl.program_id(2)
    s = jax.lax.dot_general(q_ref[...], k_ref[...], _C00,
                            preferred_element_type=jnp.float32)
    m = jnp.max(s, axis=-1, keepdims=True)
    ex = jnp.exp(s - m)
    p = ex / jnp.sum(ex, axis=-1, keepdims=True)
    pv = jax.lax.dot_general(p, v_ref[...], _C11,
                             preferred_element_type=jnp.float32)
    o = jax.lax.dot_general(pv, wot_ref[0], _C10,
                            preferred_element_type=jnp.float32)
    rows = pl.ds(i * tq, tq)

    @pl.when(h == 0)
    def _():
        out_ref[rows, :] = o
        aw_ref[0, rows, :] = p

    @pl.when(jnp.logical_and(h > 0, h < H - 1))
    def _():
        out_ref[rows, :] = out_ref[rows, :] + o
        aw_ref[0, rows, :] = aw_ref[0, rows, :] + p

    @pl.when(h == H - 1)
    def _():
        out_ref[rows, :] = out_ref[rows, :] + o + ob_ref[0]
        aw_ref[0, rows, :] = (aw_ref[0, rows, :] + p) * (1.0 / H)


def kernel(query, key, value, in_proj_weight, in_proj_bias, out_proj_w, out_proj_b,
           dgl_ln_g, dgl_ln_b, dgl_w1, dgl_b1, dgl_w2, dgl_b2):
    T, B, E = query.shape
    S = key.shape[0]
    hd = E // H
    scale = 1.0 / math.sqrt(hd)

    tt = 256
    qt, kt, vt = pl.pallas_call(
        functools.partial(_qkv_proj_kernel, n_b=B, e=E, scale=scale),
        grid=(T // tt,),
        in_specs=[
            pl.BlockSpec((tt, B, E), lambda i: (i, 0, 0)),
            pl.BlockSpec((tt, B, E), lambda i: (i, 0, 0)),
            pl.BlockSpec((tt, B, E), lambda i: (i, 0, 0)),
            pl.BlockSpec((3 * E, E), lambda i: (0, 0)),
            pl.BlockSpec((3 * E, 1), lambda i: (0, 0)),
        ],
        out_specs=[pl.BlockSpec((B * E, tt), lambda i: (0, i))] * 3,
        out_shape=[jax.ShapeDtypeStruct((B * E, T), jnp.float32)] * 3,
        compiler_params=pltpu.CompilerParams(
            dimension_semantics=("parallel",)),
    )(query, key, value, in_proj_weight, in_proj_bias.reshape(3 * E, 1))

    wot = out_proj_w.T.reshape(H, hd, E)

    tq = 256
    out, aw = pl.pallas_call(
        functools.partial(_attn_kernel, tq=tq),
        grid=(B, H, T // tq),
        in_specs=[
            pl.BlockSpec((hd, tq), lambda b, h, i: (b * H + h, i)),
            pl.BlockSpec((hd, S), lambda b, h, i: (b * H + h, 0)),
            pl.BlockSpec((hd, S), lambda b, h, i: (b * H + h, 0)),
            pl.BlockSpec((1, hd, E), lambda b, h, i: (h, 0, 0)),
            pl.BlockSpec((1, E), lambda b, h, i: (0, 0)),
        ],
        out_specs=[
            pl.BlockSpec((T, E), lambda b, h, i: (0, b)),
            pl.BlockSpec((1, T, S), lambda b, h, i: (b, 0, 0)),
        ],
        out_shape=[
            jax.ShapeDtypeStruct((T, B * E), jnp.float32),
            jax.ShapeDtypeStruct((B, T, S), jnp.float32),
        ],
        compiler_params=pltpu.CompilerParams(
            dimension_semantics=("parallel", "arbitrary", "arbitrary")),
    )(qt, kt, vt, wot, out_proj_b.reshape(1, E))

    return out.reshape(T, B, E), aw


# speedup vs baseline: 2.6867x; 1.1792x over previous
import functools
import math

import jax
import jax.numpy as jnp
from jax.experimental import pallas as pl
from jax.experimental.pallas import tpu as pltpu

H = 12

_C00 = (((0,), (0,)), ((), ()))
_C11 = (((1,), (1,)), ((), ()))
_C10 = (((1,), (0,)), ((), ()))


def _qkv_proj_kernel(xq_ref, xk_ref, xv_ref, w_ref, b_ref, qo_ref, ko_ref, vo_ref,
                     *, n_b, e, scale):
    for b in range(n_b):
        xq = xq_ref[:, b, :]
        xk = xk_ref[:, b, :]
        xv = xv_ref[:, b, :]
        rows = slice(b * e, (b + 1) * e)
        qo_ref[rows, :] = (jax.lax.dot_general(
            w_ref[:e, :], xq, _C11, preferred_element_type=jnp.float32)
            + b_ref[:e, :]) * scale
        ko_ref[rows, :] = jax.lax.dot_general(
            w_ref[e:2 * e, :], xk, _C11, preferred_element_type=jnp.float32
        ) + b_ref[e:2 * e, :]
        vo_ref[rows, :] = jax.lax.dot_general(
            w_ref[2 * e:, :], xv, _C11, preferred_element_type=jnp.float32
        ) + b_ref[2 * e:, :]


def _attn_kernel(q_ref, k_ref, v_ref, wot_ref, ob_ref, out_ref, aw_ref, *, tq):
    h = pl.program_id(1)
    i = pl.program_id(2)
    s = jax.lax.dot_general(q_ref[...], k_ref[...], _C00,
                            preferred_element_type=jnp.float32)
    m = jnp.max(s, axis=-1, keepdims=True)
    ex = jnp.exp(s - m)
    r = 1.0 / jnp.sum(ex, axis=-1, keepdims=True)
    pv = jax.lax.dot_general(ex, v_ref[...], _C11,
                             preferred_element_type=jnp.float32) * r
    o = jax.lax.dot_general(pv, wot_ref[0], _C10,
                            preferred_element_type=jnp.float32)
    rows = pl.ds(i * tq, tq)

    @pl.when(h == 0)
    def _():
        out_ref[rows, :] = o
        aw_ref[0, rows, :] = ex * r

    @pl.when(jnp.logical_and(h > 0, h < H - 1))
    def _():
        out_ref[rows, :] = out_ref[rows, :] + o
        aw_ref[0, rows, :] = aw_ref[0, rows, :] + ex * r

    @pl.when(h == H - 1)
    def _():
        out_ref[rows, :] = out_ref[rows, :] + o + ob_ref[0]
        aw_ref[0, rows, :] = (aw_ref[0, rows, :] + ex * r) * (1.0 / H)


def kernel(query, key, value, in_proj_weight, in_proj_bias, out_proj_w, out_proj_b,
           dgl_ln_g, dgl_ln_b, dgl_w1, dgl_b1, dgl_w2, dgl_b2):
    T, B, E = query.shape
    S = key.shape[0]
    hd = E // H
    scale = 1.0 / math.sqrt(hd)

    tt = 256
    qt, kt, vt = pl.pallas_call(
        functools.partial(_qkv_proj_kernel, n_b=B, e=E, scale=scale),
        grid=(T // tt,),
        in_specs=[
            pl.BlockSpec((tt, B, E), lambda i: (i, 0, 0)),
            pl.BlockSpec((tt, B, E), lambda i: (i, 0, 0)),
            pl.BlockSpec((tt, B, E), lambda i: (i, 0, 0)),
            pl.BlockSpec((3 * E, E), lambda i: (0, 0)),
            pl.BlockSpec((3 * E, 1), lambda i: (0, 0)),
        ],
        out_specs=[pl.BlockSpec((B * E, tt), lambda i: (0, i))] * 3,
        out_shape=[jax.ShapeDtypeStruct((B * E, T), jnp.float32)] * 3,
        compiler_params=pltpu.CompilerParams(
            dimension_semantics=("parallel",)),
    )(query, key, value, in_proj_weight, in_proj_bias.reshape(3 * E, 1))

    wot = out_proj_w.T.reshape(H, hd, E)

    tq = 512
    out, aw = pl.pallas_call(
        functools.partial(_attn_kernel, tq=tq),
        grid=(B, H, T // tq),
        in_specs=[
            pl.BlockSpec((hd, tq), lambda b, h, i: (b * H + h, i)),
            pl.BlockSpec((hd, S), lambda b, h, i: (b * H + h, 0)),
            pl.BlockSpec((hd, S), lambda b, h, i: (b * H + h, 0)),
            pl.BlockSpec((1, hd, E), lambda b, h, i: (h, 0, 0)),
            pl.BlockSpec((1, E), lambda b, h, i: (0, 0)),
        ],
        out_specs=[
            pl.BlockSpec((T, E), lambda b, h, i: (0, b)),
            pl.BlockSpec((1, T, S), lambda b, h, i: (b, 0, 0)),
        ],
        out_shape=[
            jax.ShapeDtypeStruct((T, B * E), jnp.float32),
            jax.ShapeDtypeStruct((B, T, S), jnp.float32),
        ],
        compiler_params=pltpu.CompilerParams(
            dimension_semantics=("parallel", "arbitrary", "arbitrary")),
    )(qt, kt, vt, wot, out_proj_b.reshape(1, E))

    return out.reshape(T, B, E), aw


# speedup vs baseline: 2.8363x; 1.0557x over previous
import functools
import math

import jax
import jax.numpy as jnp
from jax.experimental import pallas as pl
from jax.experimental.pallas import tpu as pltpu

H = 12

_C00 = (((0,), (0,)), ((), ()))
_C11 = (((1,), (1,)), ((), ()))
_C10 = (((1,), (0,)), ((), ()))


def _qkv_proj_kernel(xq_ref, xk_ref, xv_ref, w_ref, b_ref, qo_ref, ko_ref, vo_ref,
                     *, n_b, e, scale):
    for b in range(n_b):
        xq = xq_ref[:, b, :]
        xk = xk_ref[:, b, :]
        xv = xv_ref[:, b, :]
        rows = slice(b * e, (b + 1) * e)
        qo_ref[rows, :] = (jax.lax.dot_general(
            w_ref[:e, :], xq, _C11, preferred_element_type=jnp.float32)
            + b_ref[:e, :]) * scale
        ko_ref[rows, :] = jax.lax.dot_general(
            w_ref[e:2 * e, :], xk, _C11, preferred_element_type=jnp.float32
        ) + b_ref[e:2 * e, :]
        vo_ref[rows, :] = jax.lax.dot_general(
            w_ref[2 * e:, :], xv, _C11, preferred_element_type=jnp.float32
        ) + b_ref[2 * e:, :]


def _attn_kernel(q_ref, k_ref, v_ref, wot_ref, ob_ref, out_ref, aw_ref, *, tq):
    h = pl.program_id(1)
    i = pl.program_id(2)
    s = jax.lax.dot_general(q_ref[...], k_ref[...], _C00,
                            preferred_element_type=jnp.float32)
    m = jnp.max(s, axis=-1, keepdims=True)
    ex = jnp.exp(s - m)
    r = 1.0 / jnp.sum(ex, axis=-1, keepdims=True)
    pv = jax.lax.dot_general(ex, v_ref[...], _C11,
                             preferred_element_type=jnp.float32) * r
    o = jax.lax.dot_general(pv, wot_ref[0], _C10,
                            preferred_element_type=jnp.float32)
    rows = pl.ds(i * tq, tq)
    first = h == 0
    base_o = jnp.where(first, 0.0, out_ref[rows, :])
    bias = jnp.where(first, ob_ref[0], 0.0)
    out_ref[rows, :] = base_o + (o + bias)
    base_aw = jnp.where(first, 0.0, aw_ref[0, rows, :])
    aw_ref[0, rows, :] = base_aw + ex * (r * (1.0 / H))


def kernel(query, key, value, in_proj_weight, in_proj_bias, out_proj_w, out_proj_b,
           dgl_ln_g, dgl_ln_b, dgl_w1, dgl_b1, dgl_w2, dgl_b2):
    T, B, E = query.shape
    S = key.shape[0]
    hd = E // H
    scale = 1.0 / math.sqrt(hd)

    tt = 256
    qt, kt, vt = pl.pallas_call(
        functools.partial(_qkv_proj_kernel, n_b=B, e=E, scale=scale),
        grid=(T // tt,),
        in_specs=[
            pl.BlockSpec((tt, B, E), lambda i: (i, 0, 0)),
            pl.BlockSpec((tt, B, E), lambda i: (i, 0, 0)),
            pl.BlockSpec((tt, B, E), lambda i: (i, 0, 0)),
            pl.BlockSpec((3 * E, E), lambda i: (0, 0)),
            pl.BlockSpec((3 * E, 1), lambda i: (0, 0)),
        ],
        out_specs=[pl.BlockSpec((B * E, tt), lambda i: (0, i))] * 3,
        out_shape=[jax.ShapeDtypeStruct((B * E, T), jnp.float32)] * 3,
        compiler_params=pltpu.CompilerParams(
            dimension_semantics=("parallel",)),
    )(query, key, value, in_proj_weight, in_proj_bias.reshape(3 * E, 1))

    wot = out_proj_w.T.reshape(H, hd, E)

    tq = 512
    out, aw = pl.pallas_call(
        functools.partial(_attn_kernel, tq=tq),
        grid=(B, H, T // tq),
        in_specs=[
            pl.BlockSpec((hd, tq), lambda b, h, i: (b * H + h, i)),
            pl.BlockSpec((hd, S), lambda b, h, i: (b * H + h, 0)),
            pl.BlockSpec((hd, S), lambda b, h, i: (b * H + h, 0)),
            pl.BlockSpec((1, hd, E), lambda b, h, i: (h, 0, 0)),
            pl.BlockSpec((1, E), lambda b, h, i: (0, 0)),
        ],
        out_specs=[
            pl.BlockSpec((T, E), lambda b, h, i: (0, b)),
            pl.BlockSpec((1, T, S), lambda b, h, i: (b, 0, 0)),
        ],
        out_shape=[
            jax.ShapeDtypeStruct((T, B * E), jnp.float32),
            jax.ShapeDtypeStruct((B, T, S), jnp.float32),
        ],
        compiler_params=pltpu.CompilerParams(
            dimension_semantics=("parallel", "arbitrary", "arbitrary")),
    )(qt, kt, vt, wot, out_proj_b.reshape(1, E))

    return out.reshape(T, B, E), aw


# speedup vs baseline: 2.8399x; 1.0013x over previous
import functools
import math

import jax
import jax.numpy as jnp
from jax.experimental import pallas as pl
from jax.experimental.pallas import tpu as pltpu

H = 12

_C00 = (((0,), (0,)), ((), ()))
_C11 = (((1,), (1,)), ((), ()))
_C10 = (((1,), (0,)), ((), ()))


def _qkv_proj_kernel(xq_ref, xk_ref, xv_ref, w_ref, b_ref, qo_ref, ko_ref, vo_ref,
                     *, n_b, e, scale):
    for b in range(n_b):
        xq = xq_ref[:, b, :]
        xk = xk_ref[:, b, :]
        xv = xv_ref[:, b, :]
        rows = slice(b * e, (b + 1) * e)
        qo_ref[rows, :] = (jax.lax.dot_general(
            w_ref[:e, :], xq, _C11, preferred_element_type=jnp.float32)
            + b_ref[:e, :]) * scale
        ko_ref[rows, :] = jax.lax.dot_general(
            w_ref[e:2 * e, :], xk, _C11, preferred_element_type=jnp.float32
        ) + b_ref[e:2 * e, :]
        vo_ref[rows, :] = jax.lax.dot_general(
            w_ref[2 * e:, :], xv, _C11, preferred_element_type=jnp.float32
        ) + b_ref[2 * e:, :]


def _attn_kernel(q_ref, k_ref, v_ref, wot_ref, ob_ref, out_ref, aw_ref, *, tq):
    h = pl.program_id(1)
    i = pl.program_id(2)
    s = jax.lax.dot_general(q_ref[...], k_ref[...], _C00,
                            preferred_element_type=jnp.float32,
                            precision=jax.lax.Precision.DEFAULT)
    m = jnp.max(s, axis=-1, keepdims=True)
    ex = jnp.exp(s - m)
    r = 1.0 / jnp.sum(ex, axis=-1, keepdims=True)
    pv = jax.lax.dot_general(ex, v_ref[...], _C11,
                             preferred_element_type=jnp.float32,
                             precision=jax.lax.Precision.DEFAULT) * r
    o = jax.lax.dot_general(pv, wot_ref[0], _C10,
                            preferred_element_type=jnp.float32)
    rows = pl.ds(i * tq, tq)
    first = h == 0
    base_o = jnp.where(first, 0.0, out_ref[rows, :])
    bias = jnp.where(first, ob_ref[0], 0.0)
    out_ref[rows, :] = base_o + (o + bias)
    base_aw = jnp.where(first, 0.0, aw_ref[0, rows, :])
    aw_ref[0, rows, :] = base_aw + ex * (r * (1.0 / H))


def kernel(query, key, value, in_proj_weight, in_proj_bias, out_proj_w, out_proj_b,
           dgl_ln_g, dgl_ln_b, dgl_w1, dgl_b1, dgl_w2, dgl_b2):
    T, B, E = query.shape
    S = key.shape[0]
    hd = E // H
    scale = 1.0 / math.sqrt(hd)

    tt = 256
    qt, kt, vt = pl.pallas_call(
        functools.partial(_qkv_proj_kernel, n_b=B, e=E, scale=scale),
        grid=(T // tt,),
        in_specs=[
            pl.BlockSpec((tt, B, E), lambda i: (i, 0, 0)),
            pl.BlockSpec((tt, B, E), lambda i: (i, 0, 0)),
            pl.BlockSpec((tt, B, E), lambda i: (i, 0, 0)),
            pl.BlockSpec((3 * E, E), lambda i: (0, 0)),
            pl.BlockSpec((3 * E, 1), lambda i: (0, 0)),
        ],
        out_specs=[pl.BlockSpec((B * E, tt), lambda i: (0, i))] * 3,
        out_shape=[jax.ShapeDtypeStruct((B * E, T), jnp.float32)] * 3,
        compiler_params=pltpu.CompilerParams(
            dimension_semantics=("parallel",)),
    )(query, key, value, in_proj_weight, in_proj_bias.reshape(3 * E, 1))

    wot = out_proj_w.T.reshape(H, hd, E)

    tq = 512
    out, aw = pl.pallas_call(
        functools.partial(_attn_kernel, tq=tq),
        grid=(B, H, T // tq),
        in_specs=[
            pl.BlockSpec((hd, tq), lambda b, h, i: (b * H + h, i)),
            pl.BlockSpec((hd, S), lambda b, h, i: (b * H + h, 0)),
            pl.BlockSpec((hd, S), lambda b, h, i: (b * H + h, 0)),
            pl.BlockSpec((1, hd, E), lambda b, h, i: (h, 0, 0)),
            pl.BlockSpec((1, E), lambda b, h, i: (0, 0)),
        ],
        out_specs=[
            pl.BlockSpec((T, E), lambda b, h, i: (0, b)),
            pl.BlockSpec((1, T, S), lambda b, h, i: (b, 0, 0)),
        ],
        out_shape=[
            jax.ShapeDtypeStruct((T, B * E), jnp.float32),
            jax.ShapeDtypeStruct((B, T, S), jnp.float32),
        ],
        compiler_params=pltpu.CompilerParams(
            dimension_semantics=("parallel", "arbitrary", "arbitrary")),
    )(qt, kt, vt, wot, out_proj_b.reshape(1, E))

    return out.reshape(T, B, E), aw


# speedup vs baseline: 3.0575x; 1.0766x over previous
import functools
import math

import jax
import jax.numpy as jnp
from jax.experimental import pallas as pl
from jax.experimental.pallas import tpu as pltpu

H = 12

_C00 = (((0,), (0,)), ((), ()))
_C11 = (((1,), (1,)), ((), ()))
_C10 = (((1,), (0,)), ((), ()))


def _qkv_proj_kernel(xq_ref, xk_ref, xv_ref, w_ref, b_ref, qo_ref, ko_ref, vo_ref,
                     *, n_b, e, scale):
    for b in range(n_b):
        xq = xq_ref[:, b, :]
        xk = xk_ref[:, b, :]
        xv = xv_ref[:, b, :]
        rows = slice(b * e, (b + 1) * e)
        qo_ref[rows, :] = (jax.lax.dot_general(
            w_ref[:e, :], xq, _C11, preferred_element_type=jnp.float32)
            + b_ref[:e, :]) * scale
        ko_ref[rows, :] = jax.lax.dot_general(
            w_ref[e:2 * e, :], xk, _C11, preferred_element_type=jnp.float32
        ) + b_ref[e:2 * e, :]
        vo_ref[rows, :] = jax.lax.dot_general(
            w_ref[2 * e:, :], xv, _C11, preferred_element_type=jnp.float32
        ) + b_ref[2 * e:, :]


def _attn_kernel(q_ref, k_ref, v_ref, wot_ref, ob_ref, out_ref, aw_ref, *, tq):
    h = pl.program_id(1)
    i = pl.program_id(2)
    s = jax.lax.dot_general(q_ref[...], k_ref[...], _C00,
                            preferred_element_type=jnp.float32)
    m = jnp.max(s, axis=-1, keepdims=True)
    ex = jnp.exp2(s - m)
    r = 1.0 / jnp.sum(ex, axis=-1, keepdims=True)
    pv = jax.lax.dot_general(ex, v_ref[...], _C11,
                             preferred_element_type=jnp.float32) * r
    o = jax.lax.dot_general(pv, wot_ref[0], _C10,
                            preferred_element_type=jnp.float32)
    rows = pl.ds(i * tq, tq)
    first = h == 0
    base_o = jnp.where(first, 0.0, out_ref[rows, :])
    bias = jnp.where(first, ob_ref[0], 0.0)
    out_ref[rows, :] = base_o + (o + bias)
    base_aw = jnp.where(first, 0.0, aw_ref[0, rows, :])
    aw_ref[0, rows, :] = base_aw + ex * (r * (1.0 / H))


def kernel(query, key, value, in_proj_weight, in_proj_bias, out_proj_w, out_proj_b,
           dgl_ln_g, dgl_ln_b, dgl_w1, dgl_b1, dgl_w2, dgl_b2):
    T, B, E = query.shape
    S = key.shape[0]
    hd = E // H
    scale = math.log2(math.e) / math.sqrt(hd)

    tt = 256
    qt, kt, vt = pl.pallas_call(
        functools.partial(_qkv_proj_kernel, n_b=B, e=E, scale=scale),
        grid=(T // tt,),
        in_specs=[
            pl.BlockSpec((tt, B, E), lambda i: (i, 0, 0)),
            pl.BlockSpec((tt, B, E), lambda i: (i, 0, 0)),
            pl.BlockSpec((tt, B, E), lambda i: (i, 0, 0)),
            pl.BlockSpec((3 * E, E), lambda i: (0, 0)),
            pl.BlockSpec((3 * E, 1), lambda i: (0, 0)),
        ],
        out_specs=[pl.BlockSpec((B * E, tt), lambda i: (0, i))] * 3,
        out_shape=[jax.ShapeDtypeStruct((B * E, T), jnp.float32)] * 3,
        compiler_params=pltpu.CompilerParams(
            dimension_semantics=("parallel",)),
    )(query, key, value, in_proj_weight, in_proj_bias.reshape(3 * E, 1))

    wot = out_proj_w.T.reshape(H, hd, E)

    tq = 1024
    out, aw = pl.pallas_call(
        functools.partial(_attn_kernel, tq=tq),
        grid=(B, H, T // tq),
        in_specs=[
            pl.BlockSpec((hd, tq), lambda b, h, i: (b * H + h, i)),
            pl.BlockSpec((hd, S), lambda b, h, i: (b * H + h, 0)),
            pl.BlockSpec((hd, S), lambda b, h, i: (b * H + h, 0)),
            pl.BlockSpec((1, hd, E), lambda b, h, i: (h, 0, 0)),
            pl.BlockSpec((1, E), lambda b, h, i: (0, 0)),
        ],
        out_specs=[
            pl.BlockSpec((T, E), lambda b, h, i: (0, b)),
            pl.BlockSpec((1, T, S), lambda b, h, i: (b, 0, 0)),
        ],
        out_shape=[
            jax.ShapeDtypeStruct((T, B * E), jnp.float32),
            jax.ShapeDtypeStruct((B, T, S), jnp.float32),
        ],
        compiler_params=pltpu.CompilerParams(
            dimension_semantics=("parallel", "arbitrary", "arbitrary")),
    )(qt, kt, vt, wot, out_proj_b.reshape(1, E))

    return out.reshape(T, B, E), aw


# speedup vs baseline: 3.0637x; 1.0020x over previous
import functools
import math

import jax
import jax.numpy as jnp
from jax.experimental import pallas as pl
from jax.experimental.pallas import tpu as pltpu

H = 12

_C00 = (((0,), (0,)), ((), ()))
_C11 = (((1,), (1,)), ((), ()))
_C10 = (((1,), (0,)), ((), ()))


def _qkv_proj_kernel(xq_ref, xk_ref, xv_ref, w_ref, b_ref, qo_ref, ko_ref, vo_ref,
                     *, n_b, e, scale):
    for b in range(n_b):
        xq = xq_ref[:, b, :]
        xk = xk_ref[:, b, :]
        xv = xv_ref[:, b, :]
        rows = slice(b * e, (b + 1) * e)
        qo_ref[rows, :] = ((jax.lax.dot_general(
            w_ref[:e, :], xq, _C11, preferred_element_type=jnp.float32)
            + b_ref[:e, :]) * scale).astype(jnp.bfloat16)
        ko_ref[rows, :] = (jax.lax.dot_general(
            w_ref[e:2 * e, :], xk, _C11, preferred_element_type=jnp.float32
        ) + b_ref[e:2 * e, :]).astype(jnp.bfloat16)
        vo_ref[rows, :] = (jax.lax.dot_general(
            w_ref[2 * e:, :], xv, _C11, preferred_element_type=jnp.float32
        ) + b_ref[2 * e:, :]).astype(jnp.bfloat16)


def _attn_kernel(q_ref, k_ref, v_ref, wot_ref, ob_ref, out_ref, aw_ref, *, tq):
    h = pl.program_id(1)
    i = pl.program_id(2)
    s = jax.lax.dot_general(q_ref[...], k_ref[...], _C00,
                            preferred_element_type=jnp.float32)
    m = jnp.max(s, axis=-1, keepdims=True)
    ex = jnp.exp2(s - m)
    r = 1.0 / jnp.sum(ex, axis=-1, keepdims=True)
    pv = jax.lax.dot_general(ex.astype(jnp.bfloat16), v_ref[...], _C11,
                             preferred_element_type=jnp.float32) * r
    o = jax.lax.dot_general(pv, wot_ref[0], _C10,
                            preferred_element_type=jnp.float32)
    rows = pl.ds(i * tq, tq)
    first = h == 0
    base_o = jnp.where(first, 0.0, out_ref[rows, :])
    bias = jnp.where(first, ob_ref[0], 0.0)
    out_ref[rows, :] = base_o + (o + bias)
    base_aw = jnp.where(first, 0.0, aw_ref[0, rows, :])
    aw_ref[0, rows, :] = base_aw + ex * (r * (1.0 / H))


def kernel(query, key, value, in_proj_weight, in_proj_bias, out_proj_w, out_proj_b,
           dgl_ln_g, dgl_ln_b, dgl_w1, dgl_b1, dgl_w2, dgl_b2):
    T, B, E = query.shape
    S = key.shape[0]
    hd = E // H
    scale = math.log2(math.e) / math.sqrt(hd)

    tt = 256
    qt, kt, vt = pl.pallas_call(
        functools.partial(_qkv_proj_kernel, n_b=B, e=E, scale=scale),
        grid=(T // tt,),
        in_specs=[
            pl.BlockSpec((tt, B, E), lambda i: (i, 0, 0)),
            pl.BlockSpec((tt, B, E), lambda i: (i, 0, 0)),
            pl.BlockSpec((tt, B, E), lambda i: (i, 0, 0)),
            pl.BlockSpec((3 * E, E), lambda i: (0, 0)),
            pl.BlockSpec((3 * E, 1), lambda i: (0, 0)),
        ],
        out_specs=[pl.BlockSpec((B * E, tt), lambda i: (0, i))] * 3,
        out_shape=[jax.ShapeDtypeStruct((B * E, T), jnp.bfloat16)] * 3,
        compiler_params=pltpu.CompilerParams(
            dimension_semantics=("parallel",)),
    )(query, key, value, in_proj_weight, in_proj_bias.reshape(3 * E, 1))

    wot = out_proj_w.T.reshape(H, hd, E)

    tq = 1024
    out, aw = pl.pallas_call(
        functools.partial(_attn_kernel, tq=tq),
        grid=(B, H, T // tq),
        in_specs=[
            pl.BlockSpec((hd, tq), lambda b, h, i: (b * H + h, i)),
            pl.BlockSpec((hd, S), lambda b, h, i: (b * H + h, 0)),
            pl.BlockSpec((hd, S), lambda b, h, i: (b * H + h, 0)),
            pl.BlockSpec((1, hd, E), lambda b, h, i: (h, 0, 0)),
            pl.BlockSpec((1, E), lambda b, h, i: (0, 0)),
        ],
        out_specs=[
            pl.BlockSpec((T, E), lambda b, h, i: (0, b)),
            pl.BlockSpec((1, T, S), lambda b, h, i: (b, 0, 0)),
        ],
        out_shape=[
            jax.ShapeDtypeStruct((T, B * E), jnp.float32),
            jax.ShapeDtypeStruct((B, T, S), jnp.float32),
        ],
        compiler_params=pltpu.CompilerParams(
            dimension_semantics=("parallel", "arbitrary", "arbitrary")),
    )(qt, kt, vt, wot, out_proj_b.reshape(1, E))

    return out.reshape(T, B, E), aw
